# baseline (speedup 1.0000x reference)
import functools
import math

import jax
import jax.numpy as jnp
from jax import lax
from jax.experimental import pallas as pl
from jax.experimental.pallas import tpu as pltpu
from jax.experimental.pallas import tpu_sc as plsc

_S = 64.0
_M = 0.5
_COS_M = math.cos(_M)
_SIN_M = math.sin(_M)
_TH = math.cos(math.pi - _M)
_MM = math.sin(math.pi - _M) * _M

_NEG_INF = float("-inf")
_LANES = 128

_NC = 2
_NS = 16
_R_SC = 1024
_ROW_BLK = 256
_BLK_C = 1024
_ACC = 8
_SC_CW = 1408


def _exp_s(x):
    k = _S * math.log2(math.e)
    return jnp.exp2(x * k - k)


def _sc_body(cos_hbm, lab_hbm, sums_hbm, cl_hbm,
             lab_v, buf_a, buf_b, win_v, sums_v, cl_v, sem_a, sem_b, sem_w, *,
             n_cols, rpt, chunks):
    wid = lax.axis_index("s") * _NC + lax.axis_index("c")
    base = wid * rpt

    pltpu.sync_copy(lab_hbm.at[pl.ds(base, rpt)], lab_v.at[pl.ds(0, rpt)])
    lab_vecs = [lab_v[pl.ds(g * 16, 16)] for g in range(pl.cdiv(rpt, 16))]

    def lab_at(r):
        return lab_vecs[r // 16][r % 16]

    a_max = ((n_cols - 128) // 128) * 128
    lane = lax.broadcasted_iota(jnp.int32, (16,), 0)
    for p in range(pl.cdiv(rpt, 16)):
        rows_p = range(p * 16, min(rpt, p * 16 + 16))
        hs = []
        for r in rows_p:
            a_r = pl.multiple_of(
                jnp.minimum((lab_at(r) // 128) * 128, a_max), 128)
            hs.append(pltpu.async_copy(
                cos_hbm.at[pl.ds(base + (r // 8) * 8, 8), pl.ds(a_r, 128)],
                win_v.at[pl.ds((r % 16) * 8, 8)], sem_w))
        for h in hs:
            h.wait()
        for r in rows_p:
            l_r = lab_at(r)
            a_r = jnp.minimum((l_r // 128) * 128, a_max)
            off = l_r - a_r
            c_vec = jnp.full((16,), _NEG_INF, jnp.float32)
            for v in range(8):
                m = (lane + v * 16) == jnp.full((16,), off, jnp.int32)
                seg = win_v[(r % 16) * 8 + r % 8, pl.ds(v * 16, 16)]
                c_vec = jnp.maximum(c_vec, jnp.where(m, seg, _NEG_INF))
            cl_v[r] = c_vec

    zero = jnp.zeros((16,), jnp.float32)
    for r in range(rpt):
        sums_v[r] = zero

    n_chunks, cw = chunks
    rows = pl.ds(base, rpt)
    n_it = cw // (_ACC * 16)

    def start(t, buf, sem):
        return pltpu.async_copy(
            cos_hbm.at[rows, pl.ds(pl.multiple_of(t * cw, 128), cw)], buf, sem)

    def wait(t, buf, sem):
        pltpu.make_async_copy(
            cos_hbm.at[rows, pl.ds(pl.multiple_of(t * cw, 128), cw)],
            buf, sem).wait()

    def process(buf):
        def row_body(r, _):
            def col_body(it, accs):
                off = it * (_ACC * 16)
                return tuple(
                    accs[k] + jnp.exp((buf[r, pl.ds(off + k * 16, 16)] - 1.0) * _S)
                    for k in range(_ACC)
                )

            accs = lax.fori_loop(0, n_it, col_body, (zero,) * _ACC)
            tot = accs[0]
            for k in range(1, _ACC):
                tot = tot + accs[k]
            sums_v[r] = sums_v[r] + tot
            return 0

        lax.fori_loop(0, rpt, row_body, 0)

    start(0, buf_a, sem_a)
    start(1, buf_b, sem_b)

    def ring_body(g, _):
        t = 2 * g
        wait(t, buf_a, sem_a)
        process(buf_a)
        start(t + 2, buf_a, sem_a)
        wait(t + 1, buf_b, sem_b)
        process(buf_b)
        start(t + 3, buf_b, sem_b)
        return 0

    lax.fori_loop(0, (n_chunks - 3) // 2, ring_body, 0)
    wait(n_chunks - 3, buf_a, sem_a)
    process(buf_a)
    start(n_chunks - 1, buf_a, sem_a)
    wait(n_chunks - 2, buf_b, sem_b)
    process(buf_b)
    wait(n_chunks - 1, buf_a, sem_a)
    process(buf_a)

    pltpu.sync_copy(sums_v, sums_hbm.at[pl.ds(base, rpt), :])
    pltpu.sync_copy(cl_v, cl_hbm.at[pl.ds(base, rpt), :])


def _sc_part(cosine, lab1d, n_cols):
    rpt = _R_SC // (_NC * _NS)
    sc_cols = (n_cols // 128) * 128
    assert sc_cols % _SC_CW == 0 and (sc_cols // _SC_CW) % 2 == 1
    chunks = (sc_cols // _SC_CW, _SC_CW)
    mesh = plsc.VectorSubcoreMesh(core_axis_name="c", subcore_axis_name="s")
    f = pl.kernel(
        functools.partial(_sc_body, n_cols=n_cols, rpt=rpt, chunks=chunks),
        out_type=(
            jax.ShapeDtypeStruct((_R_SC, 16), jnp.float32),
            jax.ShapeDtypeStruct((_R_SC, 16), jnp.float32),
        ),
        mesh=mesh,
        scratch_types=[
            pltpu.VMEM((max(rpt, 16),), jnp.int32),
            pltpu.VMEM((rpt, _SC_CW), jnp.float32),
            pltpu.VMEM((rpt, _SC_CW), jnp.float32),
            pltpu.VMEM((min(rpt, 16) * 8, 128), jnp.float32),
            pltpu.VMEM((rpt, 16), jnp.float32),
            pltpu.VMEM((rpt, 16), jnp.float32),
            pltpu.SemaphoreType.DMA,
            pltpu.SemaphoreType.DMA,
            pltpu.SemaphoreType.DMA,
        ],
    )
    return f(cosine, lab1d)


def _tc_kernel(ca_ref, cb_ref, lab_ref, sums_ref, clout_ref, acc_ref, cl_ref,
               *, n_cb, n_cols):
    j = pl.program_id(1)
    groups = _BLK_C // _LANES

    @pl.when(j == 0)
    def _init():
        acc_ref[...] = jnp.zeros_like(acc_ref)
        cl_ref[...] = jnp.full_like(cl_ref, _NEG_INF)

    iota = lax.broadcasted_iota(jnp.int32, ca_ref.shape, 1)
    lab = lab_ref[...]

    def fold(x, blk_idx, masked):
        rel = lab - blk_idx * _BLK_C
        xl = jnp.where(iota == rel, x, _NEG_INF)
        cl = cl_ref[...]
        for g in range(groups):
            cl = jnp.maximum(cl, xl[:, g * _LANES:(g + 1) * _LANES])
        cl_ref[...] = cl

        e = _exp_s(x)
        if masked:
            e = jnp.where(blk_idx * _BLK_C + iota < n_cols, e, 0.0)
        acc = acc_ref[...]
        for g in range(groups):
            acc = acc + e[:, g * _LANES:(g + 1) * _LANES]
        acc_ref[...] = acc

    @pl.when(j < n_cb - 1)
    def _body():
        fold(ca_ref[...], 2 * j, masked=False)
        fold(cb_ref[...], 2 * j + 1, masked=False)

    @pl.when(j == n_cb - 1)
    def _tail():
        fold(ca_ref[...], 2 * j, masked=True)
        fold(cb_ref[...], 2 * j + 1, masked=True)
        sums_ref[...] = jnp.sum(acc_ref[...], axis=1, keepdims=True)
        clout_ref[...] = jnp.max(cl_ref[...], axis=1, keepdims=True)


def _tc_part(cosine, lab2d, n_cols):
    b = cosine.shape[0]
    b_tc = b - _R_SC
    n_rb = b_tc // _ROW_BLK
    rb_ofs = _R_SC // _ROW_BLK
    n_cb = pl.cdiv(pl.cdiv(n_cols, _BLK_C), 2)

    return pl.pallas_call(
        functools.partial(_tc_kernel, n_cb=n_cb, n_cols=n_cols),
        grid=(n_rb, n_cb),
        in_specs=[
            pl.BlockSpec((_ROW_BLK, _BLK_C), lambda i, j: (i + rb_ofs, 2 * j)),
            pl.BlockSpec((_ROW_BLK, _BLK_C), lambda i, j: (i + rb_ofs, 2 * j + 1)),
            pl.BlockSpec((_ROW_BLK, 1), lambda i, j: (i + rb_ofs, 0)),
        ],
        out_specs=[
            pl.BlockSpec((_ROW_BLK, 1), lambda i, j: (i, 0)),
            pl.BlockSpec((_ROW_BLK, 1), lambda i, j: (i, 0)),
        ],
        out_shape=[
            jax.ShapeDtypeStruct((b_tc, 1), jnp.float32),
            jax.ShapeDtypeStruct((b_tc, 1), jnp.float32),
        ],
        scratch_shapes=[
            pltpu.VMEM((_ROW_BLK, _LANES), jnp.float32),
            pltpu.VMEM((_ROW_BLK, _LANES), jnp.float32),
        ],
    )(cosine, cosine, lab2d)


def _row_loss(s, c_l):
    sine = jnp.sqrt(jnp.clip(1.0 - c_l * c_l, 0.0, 1.0))
    phi = c_l * _COS_M - sine * _SIN_M
    phi = jnp.where(c_l > _TH, phi, c_l - _MM)
    exp_cl = jnp.exp(_S * c_l - _S)
    exp_phi = jnp.exp(_S * phi - _S)
    s_adj = jnp.maximum(s - exp_cl, 0.0) + exp_phi
    return _S + jnp.log(s_adj) - _S * phi


def _combine_kernel(s_sc_ref, cl_sc_ref, *args, tail_start, n_cols, b):
    has_tc = len(args) == 5
    if has_tc:
        s_tc_ref, cl_tc_ref, tail_ref, lab_ref, out_ref = args
    else:
        tail_ref, lab_ref, out_ref = args
    tx = tail_ref[...]
    col = tail_start + lax.broadcasted_iota(jnp.int32, tx.shape, 1)
    e = jnp.where(col < n_cols, _exp_s(tx), 0.0)
    tail_s = jnp.sum(e, axis=1, keepdims=True)
    tail_cl = jnp.max(jnp.where(col == lab_ref[...], tx, _NEG_INF),
                      axis=1, keepdims=True)

    s_sc = jnp.sum(s_sc_ref[...], axis=1, keepdims=True) + tail_s
    cl_sc = jnp.maximum(jnp.max(cl_sc_ref[...], axis=1, keepdims=True), tail_cl)
    total = jnp.sum(_row_loss(s_sc, cl_sc))
    if has_tc:
        total = total + jnp.sum(_row_loss(s_tc_ref[...], cl_tc_ref[...]))
    out_ref[0, 0] = total / b


def _combine(s_sc, cl_sc, s_tc, cl_tc, cosine, lab_sc):
    b, c = cosine.shape
    b_tc = b - _R_SC
    tail_blk = c // 128
    specs = [
        pl.BlockSpec((_R_SC, 16), lambda i: (0, 0)),
        pl.BlockSpec((_R_SC, 16), lambda i: (0, 0)),
    ]
    ins = [s_sc, cl_sc]
    if b_tc:
        specs += [
            pl.BlockSpec((b_tc, 1), lambda i: (0, 0)),
            pl.BlockSpec((b_tc, 1), lambda i: (0, 0)),
        ]
        ins += [s_tc, cl_tc]
    specs += [
        pl.BlockSpec((_R_SC, 128), lambda i: (0, tail_blk)),
        pl.BlockSpec((_R_SC, 1), lambda i: (0, 0)),
    ]
    ins += [cosine, lab_sc]
    return pl.pallas_call(
        functools.partial(_combine_kernel, tail_start=tail_blk * 128,
                          n_cols=c, b=b),
        grid=(1,),
        in_specs=specs,
        out_specs=pl.BlockSpec(memory_space=pltpu.SMEM),
        out_shape=jax.ShapeDtypeStruct((1, 1), jnp.float32),
    )(*ins)


def kernel(cosine, label):
    b, c = cosine.shape
    lab1d = label.astype(jnp.int32)
    lab2d = lab1d.reshape(b, 1)

    sums_sc, cl_sc = _sc_part(cosine, lab1d, c)
    if b > _R_SC:
        sums_tc, cl_tc = _tc_part(cosine, lab2d, c)
    else:
        sums_tc = cl_tc = None
    return _combine(sums_sc, cl_sc, sums_tc, cl_tc, cosine, lab2d[:_R_SC])[0, 0]


# speedup vs baseline: 1.3181x; 1.3181x over previous
import functools
import math

import jax
import jax.numpy as jnp
from jax.experimental import pallas as pl
from jax.experimental.pallas import tpu as pltpu

_S = 64.0
_M = 0.5
_COS_M = math.cos(_M)
_SIN_M = math.sin(_M)
_TH = math.cos(math.pi - _M)
_MM = math.sin(math.pi - _M) * _M

_NEG_INF = float("-inf")
_LANES = 128


def _exp_s(x):
    k = _S * math.log2(math.e)
    return jnp.exp2(x * k - k)


def _loss_kernel(ca_ref, cb_ref, lab_ref, out_ref, acc_ref, cl_ref,
                 *, blk_c, n_blk, n_cols):
    j = pl.program_id(0)
    groups = blk_c // _LANES

    @pl.when(j == 0)
    def _init():
        acc_ref[...] = jnp.zeros_like(acc_ref)
        cl_ref[...] = jnp.full_like(cl_ref, _NEG_INF)

    iota = jax.lax.broadcasted_iota(jnp.int32, ca_ref.shape, 1)
    lab = lab_ref[...]

    def fold(x, blk_idx, masked):
        rel = lab - blk_idx * blk_c
        xl = jnp.where(iota == rel, x, _NEG_INF)
        cl = cl_ref[...]
        for g in range(groups):
            cl = jnp.maximum(cl, xl[:, g * _LANES:(g + 1) * _LANES])
        cl_ref[...] = cl

        e = _exp_s(x)
        if masked:
            e = jnp.where(blk_idx * blk_c + iota < n_cols, e, 0.0)
        acc = acc_ref[...]
        for g in range(groups):
            acc = acc + e[:, g * _LANES:(g + 1) * _LANES]
        acc_ref[...] = acc

    @pl.when(j < n_blk - 1)
    def _body():
        fold(ca_ref[...], 2 * j, masked=False)
        fold(cb_ref[...], 2 * j + 1, masked=False)

    @pl.when(j == n_blk - 1)
    def _tail_and_epilogue():
        fold(ca_ref[...], 2 * j, masked=True)
        fold(cb_ref[...], 2 * j + 1, masked=True)

        s = jnp.sum(acc_ref[...], axis=1, keepdims=True)
        c_l = jnp.max(cl_ref[...], axis=1, keepdims=True)
        sine = jnp.sqrt(jnp.clip(1.0 - c_l * c_l, 0.0, 1.0))
        phi = c_l * _COS_M - sine * _SIN_M
        phi = jnp.where(c_l > _TH, phi, c_l - _MM)
        exp_cl = jnp.exp(_S * c_l - _S)
        exp_phi = jnp.exp(_S * phi - _S)
        s_adj = jnp.maximum(s - exp_cl, 0.0) + exp_phi
        loss = _S + jnp.log(s_adj) - _S * phi
        out_ref[0, 0] = jnp.sum(loss) / loss.shape[0]


def kernel(cosine, label):
    b, c = cosine.shape
    blk_c = 1024
    n_blk = pl.cdiv(pl.cdiv(c, blk_c), 2)
    lab = label.astype(jnp.int32).reshape(b, 1)

    out = pl.pallas_call(
        functools.partial(_loss_kernel, blk_c=blk_c, n_blk=n_blk, n_cols=c),
        grid=(n_blk,),
        in_specs=[
            pl.BlockSpec((b, blk_c), lambda j: (0, 2 * j)),
            pl.BlockSpec((b, blk_c), lambda j: (0, 2 * j + 1)),
            pl.BlockSpec((b, 1), lambda j: (0, 0)),
        ],
        out_specs=pl.BlockSpec(memory_space=pltpu.SMEM),
        out_shape=jax.ShapeDtypeStruct((1, 1), jnp.float32),
        scratch_shapes=[
            pltpu.VMEM((b, _LANES), jnp.float32),
            pltpu.VMEM((b, _LANES), jnp.float32),
        ],
    )(cosine, cosine, lab)
    return out[0, 0]
